# baseline (speedup 1.0000x reference)
import functools

import jax
import jax.numpy as jnp
from jax import lax
from jax.experimental import pallas as pl
from jax.experimental.pallas import tpu as pltpu
from jax.experimental.pallas import tpu_sc as plsc

B, N, S, K = 8, 4096, 1024, 16
IN_CH, OUT_CH = 64, 128
EPS = 1e-5
M_TOT = B * S * K
CW = 128


def _fps_body(p_ref, ox_ref, oy_ref, oz_ref):
    P = p_ref[...]
    s0 = P[:, 0:1]
    dd = P - s0
    dd = dd * dd
    dist = (dd[0:B] + dd[B:2 * B]) + dd[2 * B:3 * B]
    iota_n = lax.broadcasted_iota(jnp.int32, (B, N), 1)
    iota_n3 = lax.broadcasted_iota(jnp.int32, (3 * B, N), 1)
    iota_c = lax.broadcasted_iota(jnp.int32, (3 * B, 128), 1)
    buf0 = jnp.where(iota_c == 0, s0, 0.0)

    def body(i, state):
        dist, buf = state
        maxv = jnp.max(dist, axis=1, keepdims=True)
        cand = jnp.where(dist == maxv, iota_n, N)
        far = jnp.min(cand, axis=1, keepdims=True)
        far3 = jnp.concatenate([far, far, far], axis=0)
        s = jnp.sum(jnp.where(iota_n3 == far3, P, 0.0), axis=1,
                    keepdims=True)
        dd = P - s
        dd = dd * dd
        d = (dd[0:B] + dd[B:2 * B]) + dd[2 * B:3 * B]
        j = lax.rem(i, 128)
        buf = jnp.where(iota_c == j, s, buf)

        @pl.when(j == 127)
        def _():
            base = pl.multiple_of(i - 127, 128)
            ox_ref[:, pl.ds(base, 128)] = buf[0:B]
            oy_ref[:, pl.ds(base, 128)] = buf[B:2 * B]
            oz_ref[:, pl.ds(base, 128)] = buf[2 * B:3 * B]

        return jnp.minimum(dist, d), buf

    lax.fori_loop(1, S, body, (dist, buf0))


def _fps(xt3):
    out = pl.pallas_call(
        _fps_body,
        out_shape=[jax.ShapeDtypeStruct((B, S), jnp.float32)] * 3,
    )(xt3)
    return out


S_BLK = 128
BIG = 1e30


def _knn_body(q_ref, p_ref, pt_ref, out_ref):
    b = pl.program_id(0)
    q = q_ref[0]
    p = p_ref[0]
    pt = pt_ref[0]
    qq = jnp.sum(q * q, axis=1, keepdims=True)
    pp = jnp.sum(pt * pt, axis=0, keepdims=True)
    qp = lax.dot_general(q.astype(jnp.bfloat16), p.astype(jnp.bfloat16),
                         (((1,), (1,)), ((), ())),
                         preferred_element_type=jnp.float32)
    d = qq + pp - 2.0 * qp

    G = 4
    W = N // G
    di = lax.bitcast_convert_type(d, jnp.int32)
    f = [(di[:, g * W:(g + 1) * W] & ~3) | g for g in range(G)]

    def cmpx(u, v):
        return jnp.minimum(u, v), jnp.maximum(u, v)

    f0, f1, f2, f3 = f
    f0, f1 = cmpx(f0, f1)
    f2, f3 = cmpx(f2, f3)
    f0, f2 = cmpx(f0, f2)
    f1, f3 = cmpx(f1, f3)
    f1, f2 = cmpx(f1, f2)

    iota = lax.broadcasted_iota(jnp.int32, (S_BLK, W), 1)
    BIGI = 0x7f7fffff
    boff = b * N
    for k in range(K):
        minv = jnp.min(f0, axis=1, keepdims=True)
        cand = jnp.where(f0 == minv, iota, W)
        pos = jnp.min(cand, axis=1, keepdims=True)
        out_ref[:, k:k + 1] = boff + (minv & 3) * W + pos
        eq = iota == pos
        f0 = jnp.where(eq, f1, f0)
        f1 = jnp.where(eq, f2, f1)
        f2 = jnp.where(eq, f3, f2)
        f3 = jnp.where(eq, BIGI, f3)


def _knn(new_xyz, xyz, xyz_t):
    return pl.pallas_call(
        _knn_body,
        grid=(B, S // S_BLK),
        in_specs=[
            pl.BlockSpec((1, S_BLK, 3), lambda b, s: (b, s, 0)),
            pl.BlockSpec((1, N, 3), lambda b, s: (b, 0, 0)),
            pl.BlockSpec((1, 3, N), lambda b, s: (b, 0, 0)),
        ],
        out_specs=pl.BlockSpec((S_BLK, K), lambda b, s: (b * (S // S_BLK) + s, 0)),
        out_shape=jax.ShapeDtypeStruct((B * S, K), jnp.int32),
    )(new_xyz, xyz, xyz_t)


NW = 32
ROWS_W = M_TOT // NW
CHUNK = 128
NCH = ROWS_W // CHUNK


def _sc_gather(table, gidx2d):
    mesh = plsc.VectorSubcoreMesh(core_axis_name="c", subcore_axis_name="s")

    @functools.partial(
        pl.kernel,
        mesh=mesh,
        out_type=jax.ShapeDtypeStruct((M_TOT, CW), jnp.float32),
        scratch_types=[
            pltpu.VMEM((NCH, CHUNK), jnp.int32),
            pltpu.VMEM((CHUNK, CW), jnp.float32),
            pltpu.SemaphoreType.DMA,
        ],
    )
    def k(table_hbm, idx_hbm, out_hbm, idx_v, rows_v, sem):
        wid = lax.axis_index("s") * 2 + lax.axis_index("c")
        base = wid * ROWS_W
        pltpu.sync_copy(idx_hbm.at[pl.ds(wid * NCH, NCH)], idx_v)

        def chunk_body(j, carry):
            pltpu.async_copy(table_hbm.at[idx_v.at[j]], rows_v, sem).wait()
            pltpu.sync_copy(rows_v, out_hbm.at[pl.ds(base + j * CHUNK, CHUNK)])
            return carry

        lax.fori_loop(0, NCH, chunk_body, 0)

    return k(table, gidx2d)


R_BLK = 1024
Q_BLK = R_BLK // K
N_STEP = M_TOT // R_BLK


def _eye_mask():
    r = lax.broadcasted_iota(jnp.int32, (CW, CW), 0)
    c = lax.broadcasted_iota(jnp.int32, (CW, CW), 1)
    return r == c


def _col67():
    c = lax.broadcasted_iota(jnp.int32, (CW, CW), 1)
    return c == (IN_CH + 3)


def _pass1_body(F_ref, Q_ref, W_ref, g_ref, b_ref, out_ref, acc_ref):
    i = pl.program_id(0)

    @pl.when(i == 0)
    def _():
        acc_ref[...] = jnp.zeros_like(acc_ref)

    Fc = (F_ref[...] - Q_ref[...]).reshape(R_BLK, CW).astype(jnp.bfloat16)
    acc_ref[...] += lax.dot_general(Fc, Fc, (((0,), (0,)), ((), ())),
                                    preferred_element_type=jnp.float32)

    @pl.when(i == pl.num_programs(0) - 1)
    def _():
        M2 = acc_ref[...]
        W = W_ref[...]
        WM = jnp.dot(W, M2, preferred_element_type=jnp.float32)
        Y2 = lax.dot_general(WM, W, (((1,), (1,)), ((), ())),
                             preferred_element_type=jnp.float32)
        ey2 = jnp.sum(jnp.where(_eye_mask(), Y2, 0.0), axis=1,
                      keepdims=True) / M_TOT
        mean = WM[:, IN_CH + 3:IN_CH + 4] / M_TOT
        var = ey2 - mean * mean
        scale = g_ref[...] / jnp.sqrt(var + EPS)
        shift = b_ref[...] - mean * scale
        out_ref[...] = W * scale + jnp.where(_col67(), shift, 0.0)


def _pass1(F3, Q3, W0aug, g0c, b0c):
    return pl.pallas_call(
        _pass1_body,
        grid=(N_STEP,),
        in_specs=[
            pl.BlockSpec((Q_BLK, K, CW), lambda i: (i, 0, 0)),
            pl.BlockSpec((Q_BLK, 1, CW), lambda i: (i, 0, 0)),
            pl.BlockSpec((CW, CW), lambda i: (0, 0)),
            pl.BlockSpec((CW, 1), lambda i: (0, 0)),
            pl.BlockSpec((CW, 1), lambda i: (0, 0)),
        ],
        out_specs=pl.BlockSpec((CW, CW), lambda i: (0, 0)),
        out_shape=jax.ShapeDtypeStruct((CW, CW), jnp.float32),
        scratch_shapes=[pltpu.VMEM((CW, CW), jnp.float32)],
    )(F3, Q3, W0aug, g0c, b0c)


def _pass2(F3, Q3, W0s, W1, g1c, b1c, beta1c):
    return pl.pallas_call(
        _pass2_body,
        grid=(N_STEP,),
        in_specs=[
            pl.BlockSpec((Q_BLK, K, CW), lambda i: (i, 0, 0)),
            pl.BlockSpec((Q_BLK, 1, CW), lambda i: (i, 0, 0)),
            pl.BlockSpec((CW, CW), lambda i: (0, 0)),
            pl.BlockSpec((CW, CW), lambda i: (0, 0)),
            pl.BlockSpec((CW, 1), lambda i: (0, 0)),
            pl.BlockSpec((CW, 1), lambda i: (0, 0)),
            pl.BlockSpec((CW, 1), lambda i: (0, 0)),
        ],
        out_specs=[
            pl.BlockSpec((CW, CW), lambda i: (0, 0)),
            pl.BlockSpec((CW, 1), lambda i: (0, 0)),
        ],
        out_shape=[
            jax.ShapeDtypeStruct((CW, CW), jnp.float32),
            jax.ShapeDtypeStruct((CW, 1), jnp.float32),
        ],
        scratch_shapes=[pltpu.VMEM((CW, CW), jnp.float32),
                        pltpu.VMEM((1, CW), jnp.float32)],
    )(F3, Q3, W0s, W1, g1c, b1c, beta1c)


def _pass2_body(F_ref, Q_ref, W0s_ref, W1_ref, g_ref, b_ref, beta_ref,
                     w_out_ref, sh_out_ref, acc_ref, sum_ref):
    i = pl.program_id(0)

    @pl.when(i == 0)
    def _():
        acc_ref[...] = jnp.zeros_like(acc_ref)
        sum_ref[...] = jnp.zeros_like(sum_ref)

    Fc = (F_ref[...] - Q_ref[...]).reshape(R_BLK, CW).astype(jnp.bfloat16)
    x1 = jnp.maximum(lax.dot_general(Fc, W0s_ref[...].astype(jnp.bfloat16),
                                     (((1,), (1,)), ((), ())),
                                     preferred_element_type=jnp.float32), 0.0)
    x1b = x1.astype(jnp.bfloat16)
    acc_ref[...] += lax.dot_general(x1b, x1b, (((0,), (0,)), ((), ())),
                                    preferred_element_type=jnp.float32)
    sum_ref[...] += jnp.sum(x1, axis=0, keepdims=True)

    @pl.when(i == pl.num_programs(0) - 1)
    def _():
        M2 = acc_ref[...]
        W1 = W1_ref[...]
        m1 = sum_ref[...] / M_TOT
        qcol = lax.dot_general(W1, m1, (((1,), (1,)), ((), ())),
                               preferred_element_type=jnp.float32)
        WM = jnp.dot(W1, M2, preferred_element_type=jnp.float32)
        Y2 = lax.dot_general(WM, W1, (((1,), (1,)), ((), ())),
                             preferred_element_type=jnp.float32)
        ey2 = jnp.sum(jnp.where(_eye_mask(), Y2, 0.0), axis=1,
                      keepdims=True) / M_TOT
        var = ey2 - qcol * qcol
        mean = qcol + b_ref[...]
        scale = g_ref[...] / jnp.sqrt(var + EPS)
        w_out_ref[...] = W1 * scale
        sh_out_ref[...] = beta_ref[...] - mean * scale


def _pass3_body(F_ref, Q_ref, W0s_ref, W1s_ref, sh_ref, out_ref):
    Fc = (F_ref[...] - Q_ref[...]).reshape(R_BLK, CW).astype(jnp.bfloat16)
    x1 = jnp.maximum(lax.dot_general(Fc, W0s_ref[...].astype(jnp.bfloat16),
                                     (((1,), (1,)), ((), ())),
                                     preferred_element_type=jnp.float32), 0.0)
    y2 = lax.dot_general(x1.astype(jnp.bfloat16),
                         W1s_ref[...].astype(jnp.bfloat16),
                         (((1,), (1,)), ((), ())),
                         preferred_element_type=jnp.float32) + sh_ref[...]
    x2 = jnp.maximum(y2, 0.0)
    out_ref[...] = jnp.max(x2.reshape(Q_BLK, K, CW), axis=1)


def _pass3(F3, Q3, W0s, W1s, sh2row):
    return pl.pallas_call(
        _pass3_body,
        grid=(N_STEP,),
        in_specs=[
            pl.BlockSpec((Q_BLK, K, CW), lambda i: (i, 0, 0)),
            pl.BlockSpec((Q_BLK, 1, CW), lambda i: (i, 0, 0)),
            pl.BlockSpec((CW, CW), lambda i: (0, 0)),
            pl.BlockSpec((CW, CW), lambda i: (0, 0)),
            pl.BlockSpec((1, CW), lambda i: (0, 0)),
        ],
        out_specs=pl.BlockSpec((Q_BLK, CW), lambda i: (i, 0)),
        out_shape=jax.ShapeDtypeStruct((B * S, CW), jnp.float32),
    )(F3, Q3, W0s, W1s, sh2row)


def kernel(xyz, features, W0, b0, gamma0, beta0, W1, b1, gamma1, beta1):
    xt = jnp.transpose(xyz, (2, 0, 1))
    ox, oy, oz = _fps(xt.reshape(3 * B, N))
    new_xyz = jnp.stack([ox, oy, oz], axis=-1)

    gidx = _knn(new_xyz, xyz, jnp.transpose(xyz, (0, 2, 1)))

    ones = jnp.ones((B, N, 1), jnp.float32)
    zpad = jnp.zeros((B, N, CW - IN_CH - 4), jnp.float32)
    table = jnp.concatenate([features, xyz, ones, zpad],
                            axis=-1).reshape(B * N, CW)

    F = _sc_gather(table, gidx.reshape(M_TOT // 128, 128))
    F3 = F.reshape(B * S, K, CW)

    q = new_xyz.reshape(B * S, 3)
    Qpad = jnp.concatenate(
        [jnp.zeros((B * S, IN_CH), jnp.float32), q,
         jnp.zeros((B * S, CW - IN_CH - 3), jnp.float32)], axis=1)
    Q3 = Qpad.reshape(B * S, 1, CW)

    W0aug = jnp.concatenate(
        [W0[:, 3:], W0[:, :3], b0[:, None],
         jnp.zeros((OUT_CH, CW - IN_CH - 4), jnp.float32)], axis=1)

    g0c = gamma0[:, None]
    b0c = beta0[:, None]
    W0s = _pass1(F3, Q3, W0aug, g0c, b0c)

    W1s, sh2 = _pass2(F3, Q3, W0s, W1, gamma1[:, None], b1[:, None],
                      beta1[:, None])
    new_feat = _pass3(F3, Q3, W0s, W1s, sh2.reshape(1, CW))
    new_features = new_feat.reshape(B, S, CW)
    return (new_xyz, new_features)


# speedup vs baseline: 1.0576x; 1.0576x over previous
import functools

import jax
import jax.numpy as jnp
from jax import lax
from jax.experimental import pallas as pl
from jax.experimental.pallas import tpu as pltpu
from jax.experimental.pallas import tpu_sc as plsc

B, N, S, K = 8, 4096, 1024, 16
IN_CH, OUT_CH = 64, 128
EPS = 1e-5
M_TOT = B * S * K
CW = 128


def _fps_body(x_ref, y_ref, z_ref, ox_ref, oy_ref, oz_ref):
    X = x_ref[...]
    Y = y_ref[...]
    Z = z_ref[...]
    x0 = X[:, 0:1]
    y0 = Y[:, 0:1]
    z0 = Z[:, 0:1]
    dx = X - x0
    dy = Y - y0
    dz = Z - z0
    dist = (dx * dx + dy * dy) + dz * dz
    iota_n = lax.broadcasted_iota(jnp.int32, (B, N), 1)
    iota_s = lax.broadcasted_iota(jnp.int32, (B, S), 1)
    rx0 = jnp.where(iota_s == 0, x0, 0.0)
    ry0 = jnp.where(iota_s == 0, y0, 0.0)
    rz0 = jnp.where(iota_s == 0, z0, 0.0)

    def body(i, state):
        dist, rx, ry, rz = state
        maxv = jnp.max(dist, axis=1, keepdims=True)
        cand = jnp.where(dist == maxv, iota_n, N)
        far = jnp.min(cand, axis=1, keepdims=True)
        onehot = iota_n == far
        xs = jnp.sum(jnp.where(onehot, X, 0.0), axis=1, keepdims=True)
        ys = jnp.sum(jnp.where(onehot, Y, 0.0), axis=1, keepdims=True)
        zs = jnp.sum(jnp.where(onehot, Z, 0.0), axis=1, keepdims=True)
        ddx = X - xs
        ddy = Y - ys
        ddz = Z - zs
        d = (ddx * ddx + ddy * ddy) + ddz * ddz
        dist = jnp.minimum(dist, d)
        sel = iota_s == i
        rx = jnp.where(sel, xs, rx)
        ry = jnp.where(sel, ys, ry)
        rz = jnp.where(sel, zs, rz)
        return dist, rx, ry, rz

    _, rx, ry, rz = lax.fori_loop(1, S, body, (dist, rx0, ry0, rz0))
    ox_ref[...] = rx
    oy_ref[...] = ry
    oz_ref[...] = rz


def _fps(xt):
    out = pl.pallas_call(
        _fps_body,
        out_shape=[jax.ShapeDtypeStruct((B, S), jnp.float32)] * 3,
    )(xt[0], xt[1], xt[2])
    return out


S_BLK = 128
BIG = 1e30


def _knn_body(q_ref, p_ref, pt_ref, out_ref):
    b = pl.program_id(0)
    q = q_ref[0]
    p = p_ref[0]
    pt = pt_ref[0]
    qq = jnp.sum(q * q, axis=1, keepdims=True)
    pp = jnp.sum(pt * pt, axis=0, keepdims=True)
    qp = lax.dot_general(q.astype(jnp.bfloat16), p.astype(jnp.bfloat16),
                         (((1,), (1,)), ((), ())),
                         preferred_element_type=jnp.float32)
    d = qq + pp - 2.0 * qp

    G = 4
    W = N // G
    di = lax.bitcast_convert_type(d, jnp.int32)
    f = [(di[:, g * W:(g + 1) * W] & ~3) | g for g in range(G)]

    def cmpx(u, v):
        return jnp.minimum(u, v), jnp.maximum(u, v)

    f0, f1, f2, f3 = f
    f0, f1 = cmpx(f0, f1)
    f2, f3 = cmpx(f2, f3)
    f0, f2 = cmpx(f0, f2)
    f1, f3 = cmpx(f1, f3)
    f1, f2 = cmpx(f1, f2)

    iota = lax.broadcasted_iota(jnp.int32, (S_BLK, W), 1)
    BIGI = 0x7f7fffff
    boff = b * N
    for k in range(K):
        minv = jnp.min(f0, axis=1, keepdims=True)
        cand = jnp.where(f0 == minv, iota, W)
        pos = jnp.min(cand, axis=1, keepdims=True)
        out_ref[:, k:k + 1] = boff + (minv & 3) * W + pos
        eq = iota == pos
        f0 = jnp.where(eq, f1, f0)
        f1 = jnp.where(eq, f2, f1)
        f2 = jnp.where(eq, f3, f2)
        f3 = jnp.where(eq, BIGI, f3)


def _knn(new_xyz, xyz, xyz_t):
    return pl.pallas_call(
        _knn_body,
        grid=(B, S // S_BLK),
        in_specs=[
            pl.BlockSpec((1, S_BLK, 3), lambda b, s: (b, s, 0)),
            pl.BlockSpec((1, N, 3), lambda b, s: (b, 0, 0)),
            pl.BlockSpec((1, 3, N), lambda b, s: (b, 0, 0)),
        ],
        out_specs=pl.BlockSpec((S_BLK, K), lambda b, s: (b * (S // S_BLK) + s, 0)),
        out_shape=jax.ShapeDtypeStruct((B * S, K), jnp.int32),
    )(new_xyz, xyz, xyz_t)


NW = 32
ROWS_W = M_TOT // NW
CHUNK = 128
NCH = ROWS_W // CHUNK


def _sc_gather(table, gidx2d):
    mesh = plsc.VectorSubcoreMesh(core_axis_name="c", subcore_axis_name="s")

    @functools.partial(
        pl.kernel,
        mesh=mesh,
        out_type=jax.ShapeDtypeStruct((M_TOT, CW), jnp.float32),
        scratch_types=[
            pltpu.VMEM((NCH, CHUNK), jnp.int32),
            pltpu.VMEM((CHUNK, CW), jnp.float32),
            pltpu.VMEM((CHUNK, CW), jnp.float32),
            pltpu.SemaphoreType.DMA,
            pltpu.SemaphoreType.DMA,
            pltpu.SemaphoreType.DMA,
            pltpu.SemaphoreType.DMA,
        ],
    )
    def k(table_hbm, idx_hbm, out_hbm, idx_v, rows0, rows1,
          gsem0, gsem1, ssem0, ssem1):
        wid = lax.axis_index("s") * 2 + lax.axis_index("c")
        base = wid * ROWS_W
        pltpu.sync_copy(idx_hbm.at[pl.ds(wid * NCH, NCH)], idx_v)

        def out_at(j):
            return out_hbm.at[pl.ds(base + j * CHUNK, CHUNK)]

        pltpu.async_copy(table_hbm.at[idx_v.at[0]], rows0, gsem0)

        def pair_body(t, carry):
            j0 = 2 * t
            j1 = j0 + 1

            @pl.when(t >= 1)
            def _():
                pltpu.make_async_copy(rows1, out_at(j1), ssem1).wait()

            pltpu.async_copy(table_hbm.at[idx_v.at[j1]], rows1, gsem1)
            pltpu.make_async_copy(table_hbm.at[idx_v.at[j0]], rows0,
                                  gsem0).wait()
            pltpu.async_copy(rows0, out_at(j0), ssem0)

            @pl.when(t <= NCH // 2 - 2)
            def _():
                pltpu.make_async_copy(rows0, out_at(j0), ssem0).wait()
                pltpu.async_copy(table_hbm.at[idx_v.at[j0 + 2]], rows0, gsem0)

            pltpu.make_async_copy(table_hbm.at[idx_v.at[j1]], rows1,
                                  gsem1).wait()
            pltpu.async_copy(rows1, out_at(j1), ssem1)
            return carry

        lax.fori_loop(0, NCH // 2, pair_body, 0)
        pltpu.make_async_copy(rows0, out_at(NCH - 2), ssem0).wait()
        pltpu.make_async_copy(rows1, out_at(NCH - 1), ssem1).wait()

    return k(table, gidx2d)


R_BLK = 1024
Q_BLK = R_BLK // K
N_STEP = M_TOT // R_BLK


def _eye_mask():
    r = lax.broadcasted_iota(jnp.int32, (CW, CW), 0)
    c = lax.broadcasted_iota(jnp.int32, (CW, CW), 1)
    return r == c


def _col67():
    c = lax.broadcasted_iota(jnp.int32, (CW, CW), 1)
    return c == (IN_CH + 3)


def _pass1_body(F_ref, Q_ref, W_ref, g_ref, b_ref, out_ref, acc_ref):
    i = pl.program_id(0)

    @pl.when(i == 0)
    def _():
        acc_ref[...] = jnp.zeros_like(acc_ref)

    Fc = (F_ref[...] - Q_ref[...]).reshape(R_BLK, CW).astype(jnp.bfloat16)
    acc_ref[...] += lax.dot_general(Fc, Fc, (((0,), (0,)), ((), ())),
                                    preferred_element_type=jnp.float32)

    @pl.when(i == pl.num_programs(0) - 1)
    def _():
        M2 = acc_ref[...]
        W = W_ref[...]
        WM = jnp.dot(W, M2, preferred_element_type=jnp.float32)
        Y2 = lax.dot_general(WM, W, (((1,), (1,)), ((), ())),
                             preferred_element_type=jnp.float32)
        ey2 = jnp.sum(jnp.where(_eye_mask(), Y2, 0.0), axis=1,
                      keepdims=True) / M_TOT
        mean = WM[:, IN_CH + 3:IN_CH + 4] / M_TOT
        var = ey2 - mean * mean
        scale = g_ref[...] / jnp.sqrt(var + EPS)
        shift = b_ref[...] - mean * scale
        out_ref[...] = W * scale + jnp.where(_col67(), shift, 0.0)


def _pass1(F3, Q3, W0aug, g0c, b0c):
    return pl.pallas_call(
        _pass1_body,
        grid=(N_STEP,),
        in_specs=[
            pl.BlockSpec((Q_BLK, K, CW), lambda i: (i, 0, 0)),
            pl.BlockSpec((Q_BLK, 1, CW), lambda i: (i, 0, 0)),
            pl.BlockSpec((CW, CW), lambda i: (0, 0)),
            pl.BlockSpec((CW, 1), lambda i: (0, 0)),
            pl.BlockSpec((CW, 1), lambda i: (0, 0)),
        ],
        out_specs=pl.BlockSpec((CW, CW), lambda i: (0, 0)),
        out_shape=jax.ShapeDtypeStruct((CW, CW), jnp.float32),
        scratch_shapes=[pltpu.VMEM((CW, CW), jnp.float32)],
    )(F3, Q3, W0aug, g0c, b0c)


def _pass2(F3, Q3, W0s, W1, g1c, b1c, beta1c):
    return pl.pallas_call(
        _pass2_body,
        grid=(N_STEP,),
        in_specs=[
            pl.BlockSpec((Q_BLK, K, CW), lambda i: (i, 0, 0)),
            pl.BlockSpec((Q_BLK, 1, CW), lambda i: (i, 0, 0)),
            pl.BlockSpec((CW, CW), lambda i: (0, 0)),
            pl.BlockSpec((CW, CW), lambda i: (0, 0)),
            pl.BlockSpec((CW, 1), lambda i: (0, 0)),
            pl.BlockSpec((CW, 1), lambda i: (0, 0)),
            pl.BlockSpec((CW, 1), lambda i: (0, 0)),
        ],
        out_specs=[
            pl.BlockSpec((CW, CW), lambda i: (0, 0)),
            pl.BlockSpec((CW, 1), lambda i: (0, 0)),
        ],
        out_shape=[
            jax.ShapeDtypeStruct((CW, CW), jnp.float32),
            jax.ShapeDtypeStruct((CW, 1), jnp.float32),
        ],
        scratch_shapes=[pltpu.VMEM((CW, CW), jnp.float32),
                        pltpu.VMEM((1, CW), jnp.float32)],
    )(F3, Q3, W0s, W1, g1c, b1c, beta1c)


def _pass2_body(F_ref, Q_ref, W0s_ref, W1_ref, g_ref, b_ref, beta_ref,
                     w_out_ref, sh_out_ref, acc_ref, sum_ref):
    i = pl.program_id(0)

    @pl.when(i == 0)
    def _():
        acc_ref[...] = jnp.zeros_like(acc_ref)
        sum_ref[...] = jnp.zeros_like(sum_ref)

    Fc = (F_ref[...] - Q_ref[...]).reshape(R_BLK, CW).astype(jnp.bfloat16)
    x1 = jnp.maximum(lax.dot_general(Fc, W0s_ref[...].astype(jnp.bfloat16),
                                     (((1,), (1,)), ((), ())),
                                     preferred_element_type=jnp.float32), 0.0)
    x1b = x1.astype(jnp.bfloat16)
    acc_ref[...] += lax.dot_general(x1b, x1b, (((0,), (0,)), ((), ())),
                                    preferred_element_type=jnp.float32)
    sum_ref[...] += jnp.sum(x1, axis=0, keepdims=True)

    @pl.when(i == pl.num_programs(0) - 1)
    def _():
        M2 = acc_ref[...]
        W1 = W1_ref[...]
        m1 = sum_ref[...] / M_TOT
        qcol = lax.dot_general(W1, m1, (((1,), (1,)), ((), ())),
                               preferred_element_type=jnp.float32)
        WM = jnp.dot(W1, M2, preferred_element_type=jnp.float32)
        Y2 = lax.dot_general(WM, W1, (((1,), (1,)), ((), ())),
                             preferred_element_type=jnp.float32)
        ey2 = jnp.sum(jnp.where(_eye_mask(), Y2, 0.0), axis=1,
                      keepdims=True) / M_TOT
        var = ey2 - qcol * qcol
        mean = qcol + b_ref[...]
        scale = g_ref[...] / jnp.sqrt(var + EPS)
        w_out_ref[...] = W1 * scale
        sh_out_ref[...] = beta_ref[...] - mean * scale


def _pass3_body(F_ref, Q_ref, W0s_ref, W1s_ref, sh_ref, out_ref):
    Fc = (F_ref[...] - Q_ref[...]).reshape(R_BLK, CW).astype(jnp.bfloat16)
    x1 = jnp.maximum(lax.dot_general(Fc, W0s_ref[...].astype(jnp.bfloat16),
                                     (((1,), (1,)), ((), ())),
                                     preferred_element_type=jnp.float32), 0.0)
    y2 = lax.dot_general(x1.astype(jnp.bfloat16),
                         W1s_ref[...].astype(jnp.bfloat16),
                         (((1,), (1,)), ((), ())),
                         preferred_element_type=jnp.float32) + sh_ref[...]
    x2 = jnp.maximum(y2, 0.0)
    out_ref[...] = jnp.max(x2.reshape(Q_BLK, K, CW), axis=1)


def _pass3(F3, Q3, W0s, W1s, sh2row):
    return pl.pallas_call(
        _pass3_body,
        grid=(N_STEP,),
        in_specs=[
            pl.BlockSpec((Q_BLK, K, CW), lambda i: (i, 0, 0)),
            pl.BlockSpec((Q_BLK, 1, CW), lambda i: (i, 0, 0)),
            pl.BlockSpec((CW, CW), lambda i: (0, 0)),
            pl.BlockSpec((CW, CW), lambda i: (0, 0)),
            pl.BlockSpec((1, CW), lambda i: (0, 0)),
        ],
        out_specs=pl.BlockSpec((Q_BLK, CW), lambda i: (i, 0)),
        out_shape=jax.ShapeDtypeStruct((B * S, CW), jnp.float32),
    )(F3, Q3, W0s, W1s, sh2row)


def kernel(xyz, features, W0, b0, gamma0, beta0, W1, b1, gamma1, beta1):
    xt = jnp.transpose(xyz, (2, 0, 1))
    ox, oy, oz = _fps(xt)
    new_xyz = jnp.stack([ox, oy, oz], axis=-1)

    gidx = _knn(new_xyz, xyz, jnp.transpose(xyz, (0, 2, 1)))

    ones = jnp.ones((B, N, 1), jnp.float32)
    zpad = jnp.zeros((B, N, CW - IN_CH - 4), jnp.float32)
    table = jnp.concatenate([features, xyz, ones, zpad],
                            axis=-1).reshape(B * N, CW)

    F = _sc_gather(table, gidx.reshape(M_TOT // 128, 128))
    F3 = F.reshape(B * S, K, CW)

    q = new_xyz.reshape(B * S, 3)
    Qpad = jnp.concatenate(
        [jnp.zeros((B * S, IN_CH), jnp.float32), q,
         jnp.zeros((B * S, CW - IN_CH - 3), jnp.float32)], axis=1)
    Q3 = Qpad.reshape(B * S, 1, CW)

    W0aug = jnp.concatenate(
        [W0[:, 3:], W0[:, :3], b0[:, None],
         jnp.zeros((OUT_CH, CW - IN_CH - 4), jnp.float32)], axis=1)

    g0c = gamma0[:, None]
    b0c = beta0[:, None]
    W0s = _pass1(F3, Q3, W0aug, g0c, b0c)

    W1s, sh2 = _pass2(F3, Q3, W0s, W1, gamma1[:, None], b1[:, None],
                      beta1[:, None])
    new_feat = _pass3(F3, Q3, W0s, W1s, sh2.reshape(1, CW))
    new_features = new_feat.reshape(B, S, CW)
    return (new_xyz, new_features)


# speedup vs baseline: 1.0976x; 1.0378x over previous
import functools

import jax
import jax.numpy as jnp
from jax import lax
from jax.experimental import pallas as pl
from jax.experimental.pallas import tpu as pltpu
from jax.experimental.pallas import tpu_sc as plsc

B, N, S, K = 8, 4096, 1024, 16
IN_CH, OUT_CH = 64, 128
EPS = 1e-5
M_TOT = B * S * K
CW = 128


def _fps_body(x_ref, y_ref, z_ref, ox_ref, oy_ref, oz_ref):
    X = x_ref[...]
    Y = y_ref[...]
    Z = z_ref[...]
    x0 = X[:, 0:1]
    y0 = Y[:, 0:1]
    z0 = Z[:, 0:1]
    dx = X - x0
    dy = Y - y0
    dz = Z - z0
    dist = (dx * dx + dy * dy) + dz * dz
    iota_n = lax.broadcasted_iota(jnp.int32, (B, N), 1)
    iota_s = lax.broadcasted_iota(jnp.int32, (B, S), 1)
    rx0 = jnp.where(iota_s == 0, x0, 0.0)
    ry0 = jnp.where(iota_s == 0, y0, 0.0)
    rz0 = jnp.where(iota_s == 0, z0, 0.0)

    def body(i, state):
        dist, rx, ry, rz = state
        maxv = jnp.max(dist, axis=1, keepdims=True)
        cand = jnp.where(dist == maxv, iota_n, N)
        far = jnp.min(cand, axis=1, keepdims=True)
        onehot = iota_n == far
        xs = jnp.sum(jnp.where(onehot, X, 0.0), axis=1, keepdims=True)
        ys = jnp.sum(jnp.where(onehot, Y, 0.0), axis=1, keepdims=True)
        zs = jnp.sum(jnp.where(onehot, Z, 0.0), axis=1, keepdims=True)
        ddx = X - xs
        ddy = Y - ys
        ddz = Z - zs
        d = (ddx * ddx + ddy * ddy) + ddz * ddz
        dist = jnp.minimum(dist, d)
        sel = iota_s == i
        rx = jnp.where(sel, xs, rx)
        ry = jnp.where(sel, ys, ry)
        rz = jnp.where(sel, zs, rz)
        return dist, rx, ry, rz

    _, rx, ry, rz = lax.fori_loop(1, S, body, (dist, rx0, ry0, rz0))
    ox_ref[...] = rx
    oy_ref[...] = ry
    oz_ref[...] = rz


def _fps(xt):
    out = pl.pallas_call(
        _fps_body,
        out_shape=[jax.ShapeDtypeStruct((B, S), jnp.float32)] * 3,
    )(xt[0], xt[1], xt[2])
    return out


S_BLK = 128
BIG = 1e30


def _knn_body(q_ref, qt_ref, p_ref, out_ref):
    b = pl.program_id(0)
    q = q_ref[0]
    qt = qt_ref[0]
    p = p_ref[0]
    qq = jnp.sum(qt * qt, axis=0, keepdims=True)
    pp = jnp.sum(p * p, axis=1, keepdims=True)
    qp = lax.dot_general(p.astype(jnp.bfloat16), q.astype(jnp.bfloat16),
                         (((1,), (1,)), ((), ())),
                         preferred_element_type=jnp.float32)
    d = (qq + pp) - 2.0 * qp

    G = 4
    W = N // G
    di = lax.bitcast_convert_type(d, jnp.int32)
    f = [(di[g * W:(g + 1) * W, :] & ~3) | g for g in range(G)]

    def cmpx(u, v):
        return jnp.minimum(u, v), jnp.maximum(u, v)

    f0, f1, f2, f3 = f
    f0, f1 = cmpx(f0, f1)
    f2, f3 = cmpx(f2, f3)
    f0, f2 = cmpx(f0, f2)
    f1, f3 = cmpx(f1, f3)
    f1, f2 = cmpx(f1, f2)

    iota = lax.broadcasted_iota(jnp.int32, (W, S_BLK), 0)
    BIGI = 0x7f7fffff
    boff = b * N

    def redmin(x):
        m = jnp.minimum(jnp.minimum(x[0:128], x[128:256]),
                        jnp.minimum(x[256:384], x[384:512]))
        m2 = jnp.minimum(jnp.minimum(x[512:640], x[640:768]),
                         jnp.minimum(x[768:896], x[896:1024]))
        return jnp.min(jnp.minimum(m, m2), axis=0, keepdims=True)

    for k in range(K):
        minv = redmin(f0)
        cand = jnp.where(f0 == minv, iota, W)
        pos = redmin(cand)
        out_ref[k:k + 1, :] = boff + (minv & 3) * W + pos
        eq = iota == pos
        f0 = jnp.where(eq, f1, f0)
        f1 = jnp.where(eq, f2, f1)
        f2 = jnp.where(eq, f3, f2)
        f3 = jnp.where(eq, BIGI, f3)


def _knn(new_xyz, new_xyz_t, xyz):
    return pl.pallas_call(
        _knn_body,
        grid=(B, S // S_BLK),
        in_specs=[
            pl.BlockSpec((1, S_BLK, 3), lambda b, s: (b, s, 0)),
            pl.BlockSpec((1, 3, S_BLK), lambda b, s: (b, 0, s)),
            pl.BlockSpec((1, N, 3), lambda b, s: (b, 0, 0)),
        ],
        out_specs=pl.BlockSpec((K, S_BLK),
                               lambda b, s: (b * (S // S_BLK) + s, 0)),
        out_shape=jax.ShapeDtypeStruct((B * (S // S_BLK) * K, S_BLK),
                                       jnp.int32),
    )(new_xyz, new_xyz_t, xyz)


NW = 32
ROWS_W = M_TOT // NW
CHUNK = 128
NCH = ROWS_W // CHUNK


def _sc_gather(table, gidx2d):
    mesh = plsc.VectorSubcoreMesh(core_axis_name="c", subcore_axis_name="s")

    @functools.partial(
        pl.kernel,
        mesh=mesh,
        out_type=jax.ShapeDtypeStruct((M_TOT, CW), jnp.float32),
        scratch_types=[
            pltpu.VMEM((NCH, CHUNK), jnp.int32),
            pltpu.VMEM((CHUNK, CW), jnp.float32),
            pltpu.VMEM((CHUNK, CW), jnp.float32),
            pltpu.SemaphoreType.DMA,
            pltpu.SemaphoreType.DMA,
            pltpu.SemaphoreType.DMA,
            pltpu.SemaphoreType.DMA,
        ],
    )
    def k(table_hbm, idx_hbm, out_hbm, idx_v, rows0, rows1,
          gsem0, gsem1, ssem0, ssem1):
        wid = lax.axis_index("s") * 2 + lax.axis_index("c")
        base = wid * ROWS_W
        pltpu.sync_copy(idx_hbm.at[pl.ds(wid * NCH, NCH)], idx_v)

        def out_at(j):
            return out_hbm.at[pl.ds(base + j * CHUNK, CHUNK)]

        pltpu.async_copy(table_hbm.at[idx_v.at[0]], rows0, gsem0)

        def pair_body(t, carry):
            j0 = 2 * t
            j1 = j0 + 1

            @pl.when(t >= 1)
            def _():
                pltpu.make_async_copy(rows1, out_at(j1), ssem1).wait()

            pltpu.async_copy(table_hbm.at[idx_v.at[j1]], rows1, gsem1)
            pltpu.make_async_copy(table_hbm.at[idx_v.at[j0]], rows0,
                                  gsem0).wait()
            pltpu.async_copy(rows0, out_at(j0), ssem0)

            @pl.when(t <= NCH // 2 - 2)
            def _():
                pltpu.make_async_copy(rows0, out_at(j0), ssem0).wait()
                pltpu.async_copy(table_hbm.at[idx_v.at[j0 + 2]], rows0, gsem0)

            pltpu.make_async_copy(table_hbm.at[idx_v.at[j1]], rows1,
                                  gsem1).wait()
            pltpu.async_copy(rows1, out_at(j1), ssem1)
            return carry

        lax.fori_loop(0, NCH // 2, pair_body, 0)
        pltpu.make_async_copy(rows0, out_at(NCH - 2), ssem0).wait()
        pltpu.make_async_copy(rows1, out_at(NCH - 1), ssem1).wait()

    return k(table, gidx2d)


R_BLK = 1024
Q_BLK = R_BLK // K
N_STEP = M_TOT // R_BLK


def _eye_mask():
    r = lax.broadcasted_iota(jnp.int32, (CW, CW), 0)
    c = lax.broadcasted_iota(jnp.int32, (CW, CW), 1)
    return r == c


def _col67():
    c = lax.broadcasted_iota(jnp.int32, (CW, CW), 1)
    return c == (IN_CH + 3)


def _pass1_body(F_ref, Q_ref, W_ref, g_ref, b_ref, out_ref, acc_ref):
    i = pl.program_id(0)

    @pl.when(i == 0)
    def _():
        acc_ref[...] = jnp.zeros_like(acc_ref)

    Fc = (F_ref[...] - Q_ref[...]).reshape(R_BLK, CW).astype(jnp.bfloat16)
    acc_ref[...] += lax.dot_general(Fc, Fc, (((0,), (0,)), ((), ())),
                                    preferred_element_type=jnp.float32)

    @pl.when(i == pl.num_programs(0) - 1)
    def _():
        M2 = acc_ref[...]
        W = W_ref[...]
        WM = jnp.dot(W, M2, preferred_element_type=jnp.float32)
        Y2 = lax.dot_general(WM, W, (((1,), (1,)), ((), ())),
                             preferred_element_type=jnp.float32)
        ey2 = jnp.sum(jnp.where(_eye_mask(), Y2, 0.0), axis=1,
                      keepdims=True) / M_TOT
        mean = WM[:, IN_CH + 3:IN_CH + 4] / M_TOT
        var = ey2 - mean * mean
        scale = g_ref[...] / jnp.sqrt(var + EPS)
        shift = b_ref[...] - mean * scale
        out_ref[...] = W * scale + jnp.where(_col67(), shift, 0.0)


def _pass1(F3, Q3, W0aug, g0c, b0c):
    return pl.pallas_call(
        _pass1_body,
        grid=(N_STEP,),
        in_specs=[
            pl.BlockSpec((Q_BLK, K, CW), lambda i: (i, 0, 0)),
            pl.BlockSpec((Q_BLK, 1, CW), lambda i: (i, 0, 0)),
            pl.BlockSpec((CW, CW), lambda i: (0, 0)),
            pl.BlockSpec((CW, 1), lambda i: (0, 0)),
            pl.BlockSpec((CW, 1), lambda i: (0, 0)),
        ],
        out_specs=pl.BlockSpec((CW, CW), lambda i: (0, 0)),
        out_shape=jax.ShapeDtypeStruct((CW, CW), jnp.float32),
        scratch_shapes=[pltpu.VMEM((CW, CW), jnp.float32)],
    )(F3, Q3, W0aug, g0c, b0c)


def _pass2(F3, Q3, W0s, W1, g1c, b1c, beta1c):
    return pl.pallas_call(
        _pass2_body,
        grid=(N_STEP,),
        in_specs=[
            pl.BlockSpec((Q_BLK, K, CW), lambda i: (i, 0, 0)),
            pl.BlockSpec((Q_BLK, 1, CW), lambda i: (i, 0, 0)),
            pl.BlockSpec((CW, CW), lambda i: (0, 0)),
            pl.BlockSpec((CW, CW), lambda i: (0, 0)),
            pl.BlockSpec((CW, 1), lambda i: (0, 0)),
            pl.BlockSpec((CW, 1), lambda i: (0, 0)),
            pl.BlockSpec((CW, 1), lambda i: (0, 0)),
        ],
        out_specs=[
            pl.BlockSpec((CW, CW), lambda i: (0, 0)),
            pl.BlockSpec((CW, 1), lambda i: (0, 0)),
        ],
        out_shape=[
            jax.ShapeDtypeStruct((CW, CW), jnp.float32),
            jax.ShapeDtypeStruct((CW, 1), jnp.float32),
        ],
        scratch_shapes=[pltpu.VMEM((CW, CW), jnp.float32),
                        pltpu.VMEM((1, CW), jnp.float32)],
    )(F3, Q3, W0s, W1, g1c, b1c, beta1c)


def _pass2_body(F_ref, Q_ref, W0s_ref, W1_ref, g_ref, b_ref, beta_ref,
                     w_out_ref, sh_out_ref, acc_ref, sum_ref):
    i = pl.program_id(0)

    @pl.when(i == 0)
    def _():
        acc_ref[...] = jnp.zeros_like(acc_ref)
        sum_ref[...] = jnp.zeros_like(sum_ref)

    Fc = (F_ref[...] - Q_ref[...]).reshape(R_BLK, CW).astype(jnp.bfloat16)
    x1 = jnp.maximum(lax.dot_general(Fc, W0s_ref[...].astype(jnp.bfloat16),
                                     (((1,), (1,)), ((), ())),
                                     preferred_element_type=jnp.float32), 0.0)
    x1b = x1.astype(jnp.bfloat16)
    acc_ref[...] += lax.dot_general(x1b, x1b, (((0,), (0,)), ((), ())),
                                    preferred_element_type=jnp.float32)
    sum_ref[...] += jnp.sum(x1, axis=0, keepdims=True)

    @pl.when(i == pl.num_programs(0) - 1)
    def _():
        M2 = acc_ref[...]
        W1 = W1_ref[...]
        m1 = sum_ref[...] / M_TOT
        qcol = lax.dot_general(W1, m1, (((1,), (1,)), ((), ())),
                               preferred_element_type=jnp.float32)
        WM = jnp.dot(W1, M2, preferred_element_type=jnp.float32)
        Y2 = lax.dot_general(WM, W1, (((1,), (1,)), ((), ())),
                             preferred_element_type=jnp.float32)
        ey2 = jnp.sum(jnp.where(_eye_mask(), Y2, 0.0), axis=1,
                      keepdims=True) / M_TOT
        var = ey2 - qcol * qcol
        mean = qcol + b_ref[...]
        scale = g_ref[...] / jnp.sqrt(var + EPS)
        w_out_ref[...] = W1 * scale
        sh_out_ref[...] = beta_ref[...] - mean * scale


def _pass3_body(F_ref, Q_ref, W0s_ref, W1s_ref, sh_ref, out_ref):
    Fc = (F_ref[...] - Q_ref[...]).reshape(R_BLK, CW).astype(jnp.bfloat16)
    x1 = jnp.maximum(lax.dot_general(Fc, W0s_ref[...].astype(jnp.bfloat16),
                                     (((1,), (1,)), ((), ())),
                                     preferred_element_type=jnp.float32), 0.0)
    y2 = lax.dot_general(x1.astype(jnp.bfloat16),
                         W1s_ref[...].astype(jnp.bfloat16),
                         (((1,), (1,)), ((), ())),
                         preferred_element_type=jnp.float32) + sh_ref[...]
    x2 = jnp.maximum(y2, 0.0)
    out_ref[...] = jnp.max(x2.reshape(Q_BLK, K, CW), axis=1)


def _pass3(F3, Q3, W0s, W1s, sh2row):
    return pl.pallas_call(
        _pass3_body,
        grid=(N_STEP,),
        in_specs=[
            pl.BlockSpec((Q_BLK, K, CW), lambda i: (i, 0, 0)),
            pl.BlockSpec((Q_BLK, 1, CW), lambda i: (i, 0, 0)),
            pl.BlockSpec((CW, CW), lambda i: (0, 0)),
            pl.BlockSpec((CW, CW), lambda i: (0, 0)),
            pl.BlockSpec((1, CW), lambda i: (0, 0)),
        ],
        out_specs=pl.BlockSpec((Q_BLK, CW), lambda i: (i, 0)),
        out_shape=jax.ShapeDtypeStruct((B * S, CW), jnp.float32),
    )(F3, Q3, W0s, W1s, sh2row)


def kernel(xyz, features, W0, b0, gamma0, beta0, W1, b1, gamma1, beta1):
    xt = jnp.transpose(xyz, (2, 0, 1))
    ox, oy, oz = _fps(xt)
    new_xyz = jnp.stack([ox, oy, oz], axis=-1)

    gidx_t = _knn(new_xyz, jnp.transpose(new_xyz, (0, 2, 1)), xyz)
    gidx = jnp.transpose(
        gidx_t.reshape(B, S // S_BLK, K, S_BLK),
        (0, 1, 3, 2)).reshape(B * S, K)

    ones = jnp.ones((B, N, 1), jnp.float32)
    zpad = jnp.zeros((B, N, CW - IN_CH - 4), jnp.float32)
    table = jnp.concatenate([features, xyz, ones, zpad],
                            axis=-1).reshape(B * N, CW)

    F = _sc_gather(table, gidx.reshape(M_TOT // 128, 128))
    F3 = F.reshape(B * S, K, CW)

    q = new_xyz.reshape(B * S, 3)
    Qpad = jnp.concatenate(
        [jnp.zeros((B * S, IN_CH), jnp.float32), q,
         jnp.zeros((B * S, CW - IN_CH - 3), jnp.float32)], axis=1)
    Q3 = Qpad.reshape(B * S, 1, CW)

    W0aug = jnp.concatenate(
        [W0[:, 3:], W0[:, :3], b0[:, None],
         jnp.zeros((OUT_CH, CW - IN_CH - 4), jnp.float32)], axis=1)

    g0c = gamma0[:, None]
    b0c = beta0[:, None]
    W0s = _pass1(F3, Q3, W0aug, g0c, b0c)

    W1s, sh2 = _pass2(F3, Q3, W0s, W1, gamma1[:, None], b1[:, None],
                      beta1[:, None])
    new_feat = _pass3(F3, Q3, W0s, W1s, sh2.reshape(1, CW))
    new_features = new_feat.reshape(B, S, CW)
    return (new_xyz, new_features)
